# Initial kernel scaffold; baseline (speedup 1.0000x reference)
import jax
import jax.numpy as jnp
from jax.experimental import pallas as pl


def kernel(x):
    raise NotImplementedError("write your pallas kernel here")


# speedup vs baseline: 11.9420x; 11.9420x over previous
import functools

import jax
import jax.numpy as jnp
from jax.experimental import pallas as pl

_K = 9
_DIL = 2
_TOPK = _K * _DIL


def _knn_block(xr_ref, xa_ref, nn_ref, cen_ref, *, rows: int, n: int):
    i = pl.program_id(1)
    xr = xr_ref[0]
    xa = xa_ref[0]

    inner = jax.lax.dot_general(
        xr, xa, (((1,), (1,)), ((), ())),
        preferred_element_type=jnp.float32,
        precision=jax.lax.Precision.DEFAULT,
    )
    sq_r = jnp.sum(xr * xr, axis=1, keepdims=True)
    sq_a = jnp.sum(xa * xa, axis=1)[None, :]
    d = sq_r - 2.0 * inner + sq_a

    col = jax.lax.broadcasted_iota(jnp.int32, (rows, n), 1)
    out_lane = jax.lax.broadcasted_iota(jnp.int32, (rows, _K), 1)
    acc = jnp.zeros((rows, _K), dtype=jnp.int32)
    for t in range(_TOPK - 1):
        m = jnp.min(d, axis=1, keepdims=True)
        idx = jnp.min(jnp.where(d == m, col, n), axis=1)
        if t % _DIL == 0:
            acc = jnp.where(out_lane == t // _DIL, idx[:, None], acc)
        if t < _TOPK - 2:
            d = jnp.where(col == idx[:, None], jnp.inf, d)

    nn_ref[0] = acc
    row0 = i * rows
    cen_ref[0] = row0 + jax.lax.broadcasted_iota(jnp.int32, (rows, _K), 0)


@jax.jit
def kernel(x):
    b, dim, n, _ = x.shape
    xt = jnp.reshape(jnp.swapaxes(x, 1, 2), (b, n, dim))
    rows = 256

    grid = (b, n // rows)
    nn, cen = pl.pallas_call(
        functools.partial(_knn_block, rows=rows, n=n),
        grid=grid,
        in_specs=[
            pl.BlockSpec((1, rows, dim), lambda bi, i: (bi, i, 0)),
            pl.BlockSpec((1, n, dim), lambda bi, i: (bi, 0, 0)),
        ],
        out_specs=[
            pl.BlockSpec((1, rows, _K), lambda bi, i: (bi, i, 0)),
            pl.BlockSpec((1, rows, _K), lambda bi, i: (bi, i, 0)),
        ],
        out_shape=[
            jax.ShapeDtypeStruct((b, n, _K), jnp.int32),
            jax.ShapeDtypeStruct((b, n, _K), jnp.int32),
        ],
    )(xt, xt)
    return jnp.stack((nn, cen), axis=0)


# speedup vs baseline: 12.8878x; 1.0792x over previous
import functools

import jax
import jax.numpy as jnp
from jax.experimental import pallas as pl

_K = 9
_DIL = 2
_TOPK = _K * _DIL
_LANES = 128


def _knn_block(xr_ref, xa_ref, nn_ref, cen_ref, *, rows: int, n: int):
    i = pl.program_id(1)
    xr = xr_ref[0]
    xa = xa_ref[0]

    inner = jax.lax.dot_general(
        xr, xa, (((1,), (1,)), ((), ())),
        preferred_element_type=jnp.float32,
        precision=jax.lax.Precision.DEFAULT,
    )
    sq_r = jnp.sum(xr * xr, axis=1, keepdims=True)
    sq_a = jnp.sum(xa * xa, axis=1)[None, :]
    d = sq_r - 2.0 * inner + sq_a

    nchunks = n // _LANES
    lanef = jax.lax.broadcasted_iota(jnp.int32, (rows, _LANES), 1).astype(
        jnp.float32)
    chunks = [d[:, c * _LANES:(c + 1) * _LANES] for c in range(nchunks)]
    inf = jnp.float32(jnp.inf)

    out_lane = jax.lax.broadcasted_iota(jnp.int32, (rows, _K), 1)
    acc = jnp.zeros((rows, _K), dtype=jnp.int32)
    prev = jnp.full((rows, 1), -1.0, dtype=jnp.float32)
    for t in range(_TOPK - 1):
        bv = None
        bi = None
        for c in range(nchunks):
            cif = lanef + jnp.float32(c * _LANES)
            cv = jnp.where(cif == prev, inf, chunks[c])
            chunks[c] = cv
            if bv is None:
                bv, bi = cv, cif
            else:
                take = cv < bv
                bv = jnp.minimum(bv, cv)
                bi = jnp.where(take, cif, bi)
        m = jnp.min(bv, axis=1, keepdims=True)
        idxf = jnp.min(jnp.where(bv == m, bi, inf), axis=1, keepdims=True)
        prev = idxf
        if t % _DIL == 0:
            idx = idxf.astype(jnp.int32)
            acc = jnp.where(out_lane == t // _DIL, idx, acc)

    nn_ref[0] = acc
    row0 = i * rows
    cen_ref[0] = row0 + jax.lax.broadcasted_iota(jnp.int32, (rows, _K), 0)


@jax.jit
def kernel(x):
    b, dim, n, _ = x.shape
    xt = jnp.reshape(jnp.swapaxes(x, 1, 2), (b, n, dim))
    rows = 256

    grid = (b, n // rows)
    nn, cen = pl.pallas_call(
        functools.partial(_knn_block, rows=rows, n=n),
        grid=grid,
        in_specs=[
            pl.BlockSpec((1, rows, dim), lambda bi, i: (bi, i, 0)),
            pl.BlockSpec((1, n, dim), lambda bi, i: (bi, 0, 0)),
        ],
        out_specs=[
            pl.BlockSpec((1, rows, _K), lambda bi, i: (bi, i, 0)),
            pl.BlockSpec((1, rows, _K), lambda bi, i: (bi, i, 0)),
        ],
        out_shape=[
            jax.ShapeDtypeStruct((b, n, _K), jnp.int32),
            jax.ShapeDtypeStruct((b, n, _K), jnp.int32),
        ],
    )(xt, xt)
    return jnp.stack((nn, cen), axis=0)


# speedup vs baseline: 12.8931x; 1.0004x over previous
import functools

import jax
import jax.numpy as jnp
from jax.experimental import pallas as pl

_K = 9
_DIL = 2
_TOPK = _K * _DIL
_LANES = 128
_RG = 64


def _knn_block(xr_ref, xa_ref, nn_ref, cen_ref, *, rows: int, n: int):
    i = pl.program_id(1)
    xr = xr_ref[0]
    xa = xa_ref[0]

    inner = jax.lax.dot_general(
        xr, xa, (((1,), (1,)), ((), ())),
        preferred_element_type=jnp.float32,
        precision=jax.lax.Precision.DEFAULT,
    )
    sq_r = jnp.sum(xr * xr, axis=1, keepdims=True)
    sq_a = jnp.sum(xa * xa, axis=1)[None, :]
    d = sq_r - 2.0 * inner + sq_a

    nchunks = n // _LANES
    colf = jax.lax.broadcasted_iota(jnp.int32, (_RG, n), 1).astype(jnp.float32)
    inf = jnp.float32(jnp.inf)
    out_lane = jax.lax.broadcasted_iota(jnp.int32, (_RG, _K), 1)

    for rg in range(rows // _RG):
        r0 = rg * _RG
        chunks = [d[r0:r0 + _RG, c * _LANES:(c + 1) * _LANES]
                  for c in range(nchunks)]
        acc = jnp.zeros((_RG, _K), dtype=jnp.int32)
        prev = jnp.full((_RG, 1), -1.0, dtype=jnp.float32)
        for t in range(_TOPK - 1):
            bv = None
            bi = None
            for c in range(nchunks):
                cif = colf[:, c * _LANES:(c + 1) * _LANES]
                cv = jnp.where(cif == prev, inf, chunks[c])
                chunks[c] = cv
                if bv is None:
                    bv, bi = cv, cif
                else:
                    take = cv < bv
                    bv = jnp.minimum(bv, cv)
                    bi = jnp.where(take, cif, bi)
            m = jnp.min(bv, axis=1, keepdims=True)
            idxf = jnp.min(jnp.where(bv == m, bi, inf), axis=1, keepdims=True)
            prev = idxf
            if t % _DIL == 0:
                idx = idxf.astype(jnp.int32)
                acc = jnp.where(out_lane == t // _DIL, idx, acc)
        nn_ref[0, r0:r0 + _RG, :] = acc
        cen_ref[0, r0:r0 + _RG, :] = (
            i * rows + r0
            + jax.lax.broadcasted_iota(jnp.int32, (_RG, _K), 0))


@jax.jit
def kernel(x):
    b, dim, n, _ = x.shape
    xt = jnp.reshape(jnp.swapaxes(x, 1, 2), (b, n, dim))
    rows = 256

    grid = (b, n // rows)
    nn, cen = pl.pallas_call(
        functools.partial(_knn_block, rows=rows, n=n),
        grid=grid,
        in_specs=[
            pl.BlockSpec((1, rows, dim), lambda bi, i: (bi, i, 0)),
            pl.BlockSpec((1, n, dim), lambda bi, i: (bi, 0, 0)),
        ],
        out_specs=[
            pl.BlockSpec((1, rows, _K), lambda bi, i: (bi, i, 0)),
            pl.BlockSpec((1, rows, _K), lambda bi, i: (bi, i, 0)),
        ],
        out_shape=[
            jax.ShapeDtypeStruct((b, n, _K), jnp.int32),
            jax.ShapeDtypeStruct((b, n, _K), jnp.int32),
        ],
    )(xt, xt)
    return jnp.stack((nn, cen), axis=0)


# speedup vs baseline: 12.9056x; 1.0010x over previous
import functools

import jax
import jax.numpy as jnp
from jax.experimental import pallas as pl
from jax.experimental.pallas import tpu as pltpu

_K = 9
_DIL = 2
_TOPK = _K * _DIL
_LANES = 128
_RG = 64


def _knn_block(xr_ref, xa_ref, nn_ref, cen_ref, *, rows: int, n: int):
    i = pl.program_id(1)
    xr = xr_ref[0]
    xa = xa_ref[0]

    inner = jax.lax.dot_general(
        xr, xa, (((1,), (1,)), ((), ())),
        preferred_element_type=jnp.float32,
        precision=jax.lax.Precision.DEFAULT,
    )
    sq_r = jnp.sum(xr * xr, axis=1, keepdims=True)
    sq_a = jnp.sum(xa * xa, axis=1)[None, :]
    d = sq_r - 2.0 * inner + sq_a

    nchunks = n // _LANES
    colf = jax.lax.broadcasted_iota(jnp.int32, (_RG, n), 1).astype(jnp.float32)
    inf = jnp.float32(jnp.inf)
    out_lane = jax.lax.broadcasted_iota(jnp.int32, (_RG, _K), 1)

    for rg in range(rows // _RG):
        r0 = rg * _RG
        chunks = [d[r0:r0 + _RG, c * _LANES:(c + 1) * _LANES]
                  for c in range(nchunks)]
        acc = jnp.zeros((_RG, _K), dtype=jnp.int32)
        prev = jnp.full((_RG, 1), -1.0, dtype=jnp.float32)
        for t in range(_TOPK - 1):
            bv = None
            bi = None
            for c in range(nchunks):
                cif = colf[:, c * _LANES:(c + 1) * _LANES]
                cv = jnp.where(cif == prev, inf, chunks[c])
                chunks[c] = cv
                if bv is None:
                    bv, bi = cv, cif
                else:
                    take = cv < bv
                    bv = jnp.minimum(bv, cv)
                    bi = jnp.where(take, cif, bi)
            m = jnp.min(bv, axis=1, keepdims=True)
            idxf = jnp.min(jnp.where(bv == m, bi, inf), axis=1, keepdims=True)
            prev = idxf
            if t % _DIL == 0:
                idx = idxf.astype(jnp.int32)
                acc = jnp.where(out_lane == t // _DIL, idx, acc)
        nn_ref[0, r0:r0 + _RG, :] = acc
        cen_ref[0, r0:r0 + _RG, :] = (
            i * rows + r0
            + jax.lax.broadcasted_iota(jnp.int32, (_RG, _K), 0))


@jax.jit
def kernel(x):
    b, dim, n, _ = x.shape
    xt = jnp.reshape(jnp.swapaxes(x, 1, 2), (b, n, dim))
    rows = 256

    grid = (b, n // rows)
    nn, cen = pl.pallas_call(
        functools.partial(_knn_block, rows=rows, n=n),
        grid=grid,
        in_specs=[
            pl.BlockSpec((1, rows, dim), lambda bi, i: (bi, i, 0)),
            pl.BlockSpec((1, n, dim), lambda bi, i: (bi, 0, 0)),
        ],
        out_specs=[
            pl.BlockSpec((1, rows, _K), lambda bi, i: (bi, i, 0)),
            pl.BlockSpec((1, rows, _K), lambda bi, i: (bi, i, 0)),
        ],
        out_shape=[
            jax.ShapeDtypeStruct((b, n, _K), jnp.int32),
            jax.ShapeDtypeStruct((b, n, _K), jnp.int32),
        ],
        compiler_params=pltpu.CompilerParams(
            dimension_semantics=("parallel", "parallel")),
    )(xt, xt)
    return jnp.stack((nn, cen), axis=0)


# speedup vs baseline: 15.0946x; 1.1696x over previous
import functools

import jax
import jax.numpy as jnp
from jax.experimental import pallas as pl
from jax.experimental.pallas import tpu as pltpu
from jax.sharding import PartitionSpec as P

_K = 9
_DIL = 2
_TOPK = _K * _DIL
_LANES = 128
_RG = 64


def _knn_block(xr_ref, xa_ref, nn_ref, cen_ref, *, rows: int, n: int):
    i = pl.program_id(1)
    xr = xr_ref[0]
    xa = xa_ref[0]

    inner = jax.lax.dot_general(
        xr, xa, (((1,), (1,)), ((), ())),
        preferred_element_type=jnp.float32,
        precision=jax.lax.Precision.DEFAULT,
    )
    sq_r = jnp.sum(xr * xr, axis=1, keepdims=True)
    sq_a = jnp.sum(xa * xa, axis=1)[None, :]
    d = sq_r - 2.0 * inner + sq_a

    nchunks = n // _LANES
    colf = jax.lax.broadcasted_iota(jnp.int32, (_RG, n), 1).astype(jnp.float32)
    inf = jnp.float32(jnp.inf)
    out_lane = jax.lax.broadcasted_iota(jnp.int32, (_RG, _K), 1)

    for rg in range(rows // _RG):
        r0 = rg * _RG
        chunks = [d[r0:r0 + _RG, c * _LANES:(c + 1) * _LANES]
                  for c in range(nchunks)]
        acc = jnp.zeros((_RG, _K), dtype=jnp.int32)
        prev = jnp.full((_RG, 1), -1.0, dtype=jnp.float32)
        for t in range(_TOPK - 1):
            bv = None
            bi = None
            for c in range(nchunks):
                cif = colf[:, c * _LANES:(c + 1) * _LANES]
                cv = jnp.where(cif == prev, inf, chunks[c])
                chunks[c] = cv
                if bv is None:
                    bv, bi = cv, cif
                else:
                    take = cv < bv
                    bv = jnp.minimum(bv, cv)
                    bi = jnp.where(take, cif, bi)
            m = jnp.min(bv, axis=1, keepdims=True)
            idxf = jnp.min(jnp.where(bv == m, bi, inf), axis=1, keepdims=True)
            prev = idxf
            if t % _DIL == 0:
                idx = idxf.astype(jnp.int32)
                acc = jnp.where(out_lane == t // _DIL, idx, acc)
        nn_ref[0, r0:r0 + _RG, :] = acc
        cen_ref[0, r0:r0 + _RG, :] = (
            i * rows + r0
            + jax.lax.broadcasted_iota(jnp.int32, (_RG, _K), 0))


def _knn_call(xt):
    b, n, dim = xt.shape
    rows = 256

    grid = (b, n // rows)
    return pl.pallas_call(
        functools.partial(_knn_block, rows=rows, n=n),
        grid=grid,
        in_specs=[
            pl.BlockSpec((1, rows, dim), lambda bi, i: (bi, i, 0)),
            pl.BlockSpec((1, n, dim), lambda bi, i: (bi, 0, 0)),
        ],
        out_specs=[
            pl.BlockSpec((1, rows, _K), lambda bi, i: (bi, i, 0)),
            pl.BlockSpec((1, rows, _K), lambda bi, i: (bi, i, 0)),
        ],
        out_shape=[
            jax.ShapeDtypeStruct((b, n, _K), jnp.int32),
            jax.ShapeDtypeStruct((b, n, _K), jnp.int32),
        ],
        compiler_params=pltpu.CompilerParams(
            dimension_semantics=("parallel", "parallel")),
    )(xt, xt)


@jax.jit
def kernel(x):
    b, dim, n, _ = x.shape
    xt = jnp.reshape(jnp.swapaxes(x, 1, 2), (b, n, dim))

    nd = jax.device_count()
    nd = 2 if (nd >= 2 and b % 2 == 0) else 1
    if nd > 1:
        mesh = jax.make_mesh((nd,), ("d",))
        xt = jax.reshard(xt, jax.NamedSharding(mesh, P("d")))
        nn, cen = jax.shard_map(
            _knn_call, mesh=mesh, in_specs=P("d"),
            out_specs=(P("d"), P("d")), check_vma=False)(xt)
    else:
        nn, cen = _knn_call(xt)
    return jnp.stack((nn, cen), axis=0)
